# baseline (speedup 1.0000x reference)
import functools

import jax
import jax.numpy as jnp
from jax import lax
from jax.experimental import pallas as pl
from jax.experimental.pallas import tpu as pltpu
from jax.experimental.pallas import tpu_sc as plsc

LATENT = 128
HIDDEN_ = 128
N_POS = 2000
N_NEG = 200
N_EDGE = N_POS + N_NEG
NC, NS = 2, 16
NW = NC * NS
POS_Q = 64
NEG_Q = 8
CHUNK = POS_Q + NEG_Q
PAD = NW * CHUNK
POS_LAST = N_POS - POS_Q
NEG_LAST = N_NEG - NEG_Q
POS_DUP = POS_Q * NW - N_POS
NEG_FULL = N_NEG // NEG_Q


@functools.cache
def _make_sc_gather():
    mesh = plsc.VectorSubcoreMesh(core_axis_name="c", subcore_axis_name="s")

    @functools.partial(
        pl.kernel,
        out_type=jax.ShapeDtypeStruct((2, NW, CHUNK, LATENT), jnp.float32),
        mesh=mesh,
        scratch_types=[
            pltpu.VMEM((2, CHUNK), jnp.int32),
            pltpu.VMEM((2, CHUNK, LATENT), jnp.float32),
            pltpu.SemaphoreType.DMA,
            pltpu.SemaphoreType.DMA,
        ],
    )
    def _sc_gather(table_hbm, edge_hbm, out_hbm, idx_v, rows_v,
                   sem_i, sem_g):
        wid = lax.axis_index("s") * NC + lax.axis_index("c")
        off_p = jnp.minimum(wid * POS_Q, POS_LAST)
        off_n = jnp.minimum(wid * NEG_Q, NEG_LAST)
        loads = []
        for h in range(2):
            loads.append(pltpu.async_copy(
                edge_hbm.at[pl.ds(h * N_EDGE + off_p, POS_Q)],
                idx_v.at[h, pl.ds(0, POS_Q)], sem_i))
            loads.append(pltpu.async_copy(
                edge_hbm.at[pl.ds(h * N_EDGE + N_POS + off_n, NEG_Q)],
                idx_v.at[h, pl.ds(POS_Q, NEG_Q)], sem_i))
        for c in loads:
            c.wait()
        gathers = [
            pltpu.async_copy(table_hbm.at[idx_v.at[h]], rows_v.at[h], sem_g)
            for h in range(2)
        ]
        for h in range(2):
            gathers[h].wait()
            pltpu.sync_copy(rows_v.at[h], out_hbm.at[h, wid])

    return _sc_gather


NB = 4
BLK = PAD // NB


def _tc_mlp_loss(g_ref, w1_ref, b1_ref, w2_ref, b2_ref, out_ref):
    i = pl.program_id(0)
    src = g_ref[0]
    tgt = g_ref[1]
    w1 = w1_ref[...]
    h = lax.dot_general(src, w1[:, :LATENT], (((1,), (1,)), ((), ())),
                        preferred_element_type=jnp.float32)
    h = h + lax.dot_general(tgt, w1[:, LATENT:], (((1,), (1,)), ((), ())),
                            preferred_element_type=jnp.float32)
    h = jnp.maximum(h + b1_ref[...], 0.0)
    w2b = jnp.broadcast_to(w2_ref[...], (LATENT, LATENT))
    s = lax.dot_general(h, w2b, (((1,), (1,)), ((), ())),
                        preferred_element_type=jnp.float32)
    s = s + b2_ref[0, 0]
    rows = i * BLK + lax.broadcasted_iota(jnp.int32, (BLK, LATENT), 0)
    w = rows // CHUNK
    k = rows - w * CHUNK
    is_pos = k < POS_Q
    valid = jnp.logical_or(
        jnp.logical_and(is_pos, jnp.logical_or(w < NW - 1, k >= POS_DUP)),
        jnp.logical_and(jnp.logical_not(is_pos), w < NEG_FULL),
    )
    label = is_pos.astype(jnp.float32)
    per = jnp.maximum(s, 0.0) - s * label + jnp.log1p(jnp.exp(-jnp.abs(s)))
    per = jnp.where(valid, per, 0.0)
    part = jnp.sum(per) * (1.0 / (N_EDGE * LATENT))

    @pl.when(i == 0)
    def _():
        out_ref[0, 0] = 0.0

    out_ref[0, 0] += part


def kernel(v_gene, pos_edge_index, neg_edge_index, W1, b1, W2, b2):
    edges = jnp.concatenate(
        [pos_edge_index.astype(jnp.int32), neg_edge_index.astype(jnp.int32)],
        axis=1).reshape(-1)
    gathered = _make_sc_gather()(v_gene, edges)
    g = gathered.reshape(2, PAD, LATENT)

    loss = pl.pallas_call(
        _tc_mlp_loss,
        grid=(NB,),
        out_shape=jax.ShapeDtypeStruct((1, 1), jnp.float32),
        in_specs=[
            pl.BlockSpec((2, BLK, LATENT), lambda i: (0, i, 0)),
            pl.BlockSpec((HIDDEN_, 2 * LATENT), lambda i: (0, 0)),
            pl.BlockSpec((1, LATENT), lambda i: (0, 0)),
            pl.BlockSpec((1, HIDDEN_), lambda i: (0, 0)),
            pl.BlockSpec(memory_space=pltpu.SMEM),
        ],
        out_specs=pl.BlockSpec(memory_space=pltpu.SMEM),
    )(g, W1, b1.reshape(1, LATENT), W2, b2.reshape(1, 1))
    return loss[0, 0]


# speedup vs baseline: 1.0585x; 1.0585x over previous
import functools

import jax
import jax.numpy as jnp
from jax import lax
from jax.experimental import pallas as pl
from jax.experimental.pallas import tpu as pltpu
from jax.experimental.pallas import tpu_sc as plsc

LATENT = 128
HIDDEN_ = 128
N_POS = 2000
N_NEG = 200
N_EDGE = N_POS + N_NEG
NC, NS = 2, 16
NW = NC * NS
POS_Q = 64
NEG_Q = 8
CHUNK = POS_Q + NEG_Q
PAD = NW * CHUNK
POS_LAST = N_POS - POS_Q
NEG_LAST = N_NEG - NEG_Q
POS_DUP = POS_Q * NW - N_POS
NEG_FULL = N_NEG // NEG_Q


@functools.cache
def _make_sc_gather():
    mesh = plsc.VectorSubcoreMesh(core_axis_name="c", subcore_axis_name="s")

    @functools.partial(
        pl.kernel,
        out_type=jax.ShapeDtypeStruct((2, NW, CHUNK, LATENT), jnp.float32),
        mesh=mesh,
        scratch_types=[
            pltpu.VMEM((2, CHUNK), jnp.int32),
            pltpu.VMEM((2, CHUNK, LATENT), jnp.float32),
            pltpu.SemaphoreType.DMA,
            pltpu.SemaphoreType.DMA,
        ],
    )
    def _sc_gather(table_hbm, edge_hbm, out_hbm, idx_v, rows_v,
                   sem_i, sem_g):
        wid = lax.axis_index("s") * NC + lax.axis_index("c")
        off_p = jnp.minimum(wid * POS_Q, POS_LAST)
        off_n = jnp.minimum(wid * NEG_Q, NEG_LAST)
        loads = []
        for h in range(2):
            loads.append(pltpu.async_copy(
                edge_hbm.at[pl.ds(h * N_EDGE + off_p, POS_Q)],
                idx_v.at[h, pl.ds(0, POS_Q)], sem_i))
            loads.append(pltpu.async_copy(
                edge_hbm.at[pl.ds(h * N_EDGE + N_POS + off_n, NEG_Q)],
                idx_v.at[h, pl.ds(POS_Q, NEG_Q)], sem_i))
        for c in loads:
            c.wait()
        gathers = [
            pltpu.async_copy(table_hbm.at[idx_v.at[h]], rows_v.at[h], sem_g)
            for h in range(2)
        ]
        for h in range(2):
            gathers[h].wait()
            pltpu.sync_copy(rows_v.at[h], out_hbm.at[h, wid])

    return _sc_gather


def _tc_mlp_loss(g_ref, w1_ref, b1_ref, w2_ref, b2_ref, out_ref):
    src = g_ref[0]
    tgt = g_ref[1]
    w1 = w1_ref[...]
    h = lax.dot_general(src, w1[:, :LATENT], (((1,), (1,)), ((), ())),
                        preferred_element_type=jnp.float32)
    h = h + lax.dot_general(tgt, w1[:, LATENT:], (((1,), (1,)), ((), ())),
                            preferred_element_type=jnp.float32)
    h = jnp.maximum(h + b1_ref[...], 0.0)
    w2b = jnp.broadcast_to(w2_ref[...], (LATENT, LATENT))
    s = lax.dot_general(h, w2b, (((1,), (1,)), ((), ())),
                        preferred_element_type=jnp.float32)
    s = s + b2_ref[0, 0]
    rows = lax.broadcasted_iota(jnp.int32, (PAD, LATENT), 0)
    w = rows // CHUNK
    k = rows - w * CHUNK
    is_pos = k < POS_Q
    valid = jnp.logical_or(
        jnp.logical_and(is_pos, jnp.logical_or(w < NW - 1, k >= POS_DUP)),
        jnp.logical_and(jnp.logical_not(is_pos), w < NEG_FULL),
    )
    label = is_pos.astype(jnp.float32)
    per = jnp.maximum(s, 0.0) - s * label + jnp.log1p(jnp.exp(-jnp.abs(s)))
    per = jnp.where(valid, per, 0.0)
    out_ref[0, 0] = jnp.sum(per) * (1.0 / (N_EDGE * LATENT))


def kernel(v_gene, pos_edge_index, neg_edge_index, W1, b1, W2, b2):
    edges = jnp.concatenate(
        [pos_edge_index.astype(jnp.int32), neg_edge_index.astype(jnp.int32)],
        axis=1).reshape(-1)
    gathered = _make_sc_gather()(v_gene, edges)
    g = gathered.reshape(2, PAD, LATENT)

    loss = pl.pallas_call(
        _tc_mlp_loss,
        out_shape=jax.ShapeDtypeStruct((1, 1), jnp.float32),
        in_specs=[
            pl.BlockSpec(memory_space=pltpu.VMEM),
            pl.BlockSpec(memory_space=pltpu.VMEM),
            pl.BlockSpec(memory_space=pltpu.VMEM),
            pl.BlockSpec(memory_space=pltpu.VMEM),
            pl.BlockSpec(memory_space=pltpu.SMEM),
        ],
        out_specs=pl.BlockSpec(memory_space=pltpu.SMEM),
    )(g, W1, b1.reshape(1, LATENT), W2, b2.reshape(1, 1))
    return loss[0, 0]


# speedup vs baseline: 1.0981x; 1.0374x over previous
import functools

import jax
import jax.numpy as jnp
from jax import lax
from jax.experimental import pallas as pl
from jax.experimental.pallas import tpu as pltpu
from jax.experimental.pallas import tpu_sc as plsc

LATENT = 128
HIDDEN_ = 128
N_POS = 2000
N_NEG = 200
N_EDGE = N_POS + N_NEG
NC, NS = 2, 16
NW = NC * NS
POS_Q = 64
NEG_Q = 8
CHUNK = POS_Q + NEG_Q
PAD = NW * CHUNK
POS_LAST = N_POS - POS_Q
NEG_LAST = N_NEG - NEG_Q
POS_DUP = POS_Q * NW - N_POS
NEG_FULL = N_NEG // NEG_Q


@functools.cache
def _make_sc_gather():
    mesh = plsc.VectorSubcoreMesh(core_axis_name="c", subcore_axis_name="s")

    @functools.partial(
        pl.kernel,
        out_type=jax.ShapeDtypeStruct((2, NW, CHUNK, LATENT), jnp.float32),
        mesh=mesh,
        scratch_types=[
            pltpu.VMEM((2, CHUNK), jnp.int32),
            pltpu.VMEM((2, CHUNK, LATENT), jnp.float32),
            pltpu.SemaphoreType.DMA,
            pltpu.SemaphoreType.DMA,
        ],
    )
    def _sc_gather(table_hbm, edge_hbm, out_hbm, idx_v, rows_v,
                   sem_i, sem_g):
        wid = lax.axis_index("s") * NC + lax.axis_index("c")
        off_p = jnp.minimum(wid * POS_Q, POS_LAST)
        off_n = jnp.minimum(wid * NEG_Q, NEG_LAST)
        loads = []
        for h in range(2):
            loads.append(pltpu.async_copy(
                edge_hbm.at[pl.ds(h * N_EDGE + off_p, POS_Q)],
                idx_v.at[h, pl.ds(0, POS_Q)], sem_i))
            loads.append(pltpu.async_copy(
                edge_hbm.at[pl.ds(h * N_EDGE + N_POS + off_n, NEG_Q)],
                idx_v.at[h, pl.ds(POS_Q, NEG_Q)], sem_i))
        for c in loads:
            c.wait()
        gathers = [
            pltpu.async_copy(table_hbm.at[idx_v.at[h]], rows_v.at[h], sem_g)
            for h in range(2)
        ]
        for h in range(2):
            gathers[h].wait()
            pltpu.sync_copy(rows_v.at[h], out_hbm.at[h, wid])

    return _sc_gather


def _tc_mlp_loss(g_ref, w1_ref, b1_ref, w2_ref, b2_ref, out_ref):
    src = g_ref[0]
    tgt = g_ref[1]
    w1 = w1_ref[...]
    h = lax.dot_general(src, w1[:, :LATENT], (((1,), (1,)), ((), ())),
                        preferred_element_type=jnp.float32)
    h = h + lax.dot_general(tgt, w1[:, LATENT:], (((1,), (1,)), ((), ())),
                            preferred_element_type=jnp.float32)
    h = jnp.maximum(h + b1_ref[...], 0.0)
    s = lax.dot_general(w2_ref[...], h, (((1,), (1,)), ((), ())),
                        preferred_element_type=jnp.float32)
    s = s + b2_ref[0, 0]
    rows = lax.broadcasted_iota(jnp.int32, (1, PAD), 1)
    w = rows // CHUNK
    k = rows - w * CHUNK
    is_pos = k < POS_Q
    valid = jnp.logical_or(
        jnp.logical_and(is_pos, jnp.logical_or(w < NW - 1, k >= POS_DUP)),
        jnp.logical_and(jnp.logical_not(is_pos), w < NEG_FULL),
    )
    label = is_pos.astype(jnp.float32)
    per = jnp.maximum(s, 0.0) - s * label + jnp.log1p(jnp.exp(-jnp.abs(s)))
    per = jnp.where(valid, per, 0.0)
    out_ref[0, 0] = jnp.sum(per) * (1.0 / N_EDGE)


def kernel(v_gene, pos_edge_index, neg_edge_index, W1, b1, W2, b2):
    edges = jnp.concatenate(
        [pos_edge_index.astype(jnp.int32), neg_edge_index.astype(jnp.int32)],
        axis=1).reshape(-1)
    gathered = _make_sc_gather()(v_gene, edges)
    g = gathered.reshape(2, PAD, LATENT)

    loss = pl.pallas_call(
        _tc_mlp_loss,
        out_shape=jax.ShapeDtypeStruct((1, 1), jnp.float32),
        in_specs=[
            pl.BlockSpec(memory_space=pltpu.VMEM),
            pl.BlockSpec(memory_space=pltpu.VMEM),
            pl.BlockSpec(memory_space=pltpu.VMEM),
            pl.BlockSpec(memory_space=pltpu.VMEM),
            pl.BlockSpec(memory_space=pltpu.SMEM),
        ],
        out_specs=pl.BlockSpec(memory_space=pltpu.SMEM),
    )(g, W1, b1.reshape(1, LATENT), W2, b2.reshape(1, 1))
    return loss[0, 0]


# speedup vs baseline: 1.1026x; 1.0042x over previous
import functools

import jax
import jax.numpy as jnp
from jax import lax
from jax.experimental import pallas as pl
from jax.experimental.pallas import tpu as pltpu
from jax.experimental.pallas import tpu_sc as plsc

LATENT = 128
HIDDEN_ = 128
N_POS = 2000
N_NEG = 200
N_EDGE = N_POS + N_NEG
NC, NS = 2, 16
NW = NC * NS
POS_Q = 64
NEG_Q = 8
CHUNK = POS_Q + NEG_Q
PAD = NW * CHUNK
POS_LAST = N_POS - POS_Q
NEG_LAST = N_NEG - NEG_Q
POS_DUP = POS_Q * NW - N_POS
NEG_FULL = N_NEG // NEG_Q


@functools.cache
def _make_sc_gather():
    mesh = plsc.VectorSubcoreMesh(core_axis_name="c", subcore_axis_name="s")

    @functools.partial(
        pl.kernel,
        out_type=jax.ShapeDtypeStruct((2, NW, CHUNK, LATENT), jnp.float32),
        mesh=mesh,
        scratch_types=[
            pltpu.VMEM((2, CHUNK), jnp.int32),
            pltpu.VMEM((2, CHUNK, LATENT), jnp.float32),
            pltpu.SemaphoreType.DMA,
            pltpu.SemaphoreType.DMA,
        ],
    )
    def _sc_gather(table_hbm, edge_hbm, out_hbm, idx_v, rows_v,
                   sem_i, sem_g):
        wid = lax.axis_index("s") * NC + lax.axis_index("c")
        off_p = jnp.minimum(wid * POS_Q, POS_LAST)
        off_n = jnp.minimum(wid * NEG_Q, NEG_LAST)
        loads = []
        for h in range(2):
            loads.append(pltpu.async_copy(
                edge_hbm.at[pl.ds(h * N_EDGE + off_p, POS_Q)],
                idx_v.at[h, pl.ds(0, POS_Q)], sem_i))
            loads.append(pltpu.async_copy(
                edge_hbm.at[pl.ds(h * N_EDGE + N_POS + off_n, NEG_Q)],
                idx_v.at[h, pl.ds(POS_Q, NEG_Q)], sem_i))
        for c in loads:
            c.wait()
        gathers = [
            pltpu.async_copy(table_hbm.at[idx_v.at[h]], rows_v.at[h], sem_g)
            for h in range(2)
        ]
        writes = []
        for h in range(2):
            gathers[h].wait()
            writes.append(pltpu.async_copy(rows_v.at[h], out_hbm.at[h, wid],
                                           sem_i))
        for c in writes:
            c.wait()

    return _sc_gather


def _tc_mlp_loss(g_ref, w1_ref, b1_ref, w2_ref, b2_ref, out_ref):
    src = g_ref[0]
    tgt = g_ref[1]
    w1 = w1_ref[...]
    h = lax.dot_general(src, w1[:, :LATENT], (((1,), (1,)), ((), ())),
                        preferred_element_type=jnp.float32)
    h = h + lax.dot_general(tgt, w1[:, LATENT:], (((1,), (1,)), ((), ())),
                            preferred_element_type=jnp.float32)
    h = jnp.maximum(h + b1_ref[...], 0.0)
    s = lax.dot_general(w2_ref[...], h, (((1,), (1,)), ((), ())),
                        preferred_element_type=jnp.float32)
    s = s + b2_ref[0, 0]
    rows = lax.broadcasted_iota(jnp.int32, (1, PAD), 1)
    w = rows // CHUNK
    k = rows - w * CHUNK
    is_pos = k < POS_Q
    valid = jnp.logical_or(
        jnp.logical_and(is_pos, jnp.logical_or(w < NW - 1, k >= POS_DUP)),
        jnp.logical_and(jnp.logical_not(is_pos), w < NEG_FULL),
    )
    label = is_pos.astype(jnp.float32)
    per = jnp.maximum(s, 0.0) - s * label + jnp.log1p(jnp.exp(-jnp.abs(s)))
    per = jnp.where(valid, per, 0.0)
    out_ref[0, 0] = jnp.sum(per) * (1.0 / N_EDGE)


def kernel(v_gene, pos_edge_index, neg_edge_index, W1, b1, W2, b2):
    edges = jnp.concatenate(
        [pos_edge_index.astype(jnp.int32), neg_edge_index.astype(jnp.int32)],
        axis=1).reshape(-1)
    gathered = _make_sc_gather()(v_gene, edges)
    g = gathered.reshape(2, PAD, LATENT)

    loss = pl.pallas_call(
        _tc_mlp_loss,
        out_shape=jax.ShapeDtypeStruct((1, 1), jnp.float32),
        in_specs=[
            pl.BlockSpec(memory_space=pltpu.VMEM),
            pl.BlockSpec(memory_space=pltpu.VMEM),
            pl.BlockSpec(memory_space=pltpu.VMEM),
            pl.BlockSpec(memory_space=pltpu.VMEM),
            pl.BlockSpec(memory_space=pltpu.SMEM),
        ],
        out_specs=pl.BlockSpec(memory_space=pltpu.SMEM),
    )(g, W1, b1.reshape(1, LATENT), W2, b2.reshape(1, 1))
    return loss[0, 0]


# speedup vs baseline: 1.1027x; 1.0001x over previous
import functools

import jax
import jax.numpy as jnp
from jax import lax
from jax.experimental import pallas as pl
from jax.experimental.pallas import tpu as pltpu
from jax.experimental.pallas import tpu_sc as plsc

LATENT = 128
HIDDEN_ = 128
N_POS = 2000
N_NEG = 200
N_EDGE = N_POS + N_NEG
NC, NS = 2, 16
NW = NC * NS
POS_Q = 64
NEG_Q = 8
CHUNK = POS_Q + NEG_Q
PAD = NW * CHUNK
POS_LAST = N_POS - POS_Q
NEG_LAST = N_NEG - NEG_Q
POS_DUP = POS_Q * NW - N_POS
NEG_FULL = N_NEG // NEG_Q


@functools.cache
def _make_sc_gather():
    mesh = plsc.VectorSubcoreMesh(core_axis_name="c", subcore_axis_name="s")

    @functools.partial(
        pl.kernel,
        out_type=jax.ShapeDtypeStruct((2, NW, CHUNK, LATENT), jnp.float32),
        mesh=mesh,
        scratch_types=[
            pltpu.VMEM((2, CHUNK), jnp.int32),
            pltpu.VMEM((2, CHUNK, LATENT), jnp.float32),
            pltpu.SemaphoreType.DMA,
            pltpu.SemaphoreType.DMA,
        ],
    )
    def _sc_gather(table_hbm, edge_hbm, out_hbm, idx_v, rows_v,
                   sem_i, sem_g):
        wid = lax.axis_index("s") * NC + lax.axis_index("c")
        off_p = jnp.minimum(wid * POS_Q, POS_LAST)
        off_n = jnp.minimum(wid * NEG_Q, NEG_LAST)
        loads = []
        for h in range(2):
            loads.append(pltpu.async_copy(
                edge_hbm.at[pl.ds(h * N_EDGE + off_p, POS_Q)],
                idx_v.at[h, pl.ds(0, POS_Q)], sem_i))
            loads.append(pltpu.async_copy(
                edge_hbm.at[pl.ds(h * N_EDGE + N_POS + off_n, NEG_Q)],
                idx_v.at[h, pl.ds(POS_Q, NEG_Q)], sem_i))
        for c in loads:
            c.wait()
        gathers = [
            pltpu.async_copy(table_hbm.at[idx_v.at[h]], rows_v.at[h], sem_g)
            for h in range(2)
        ]
        for g in gathers:
            g.wait()
        pltpu.sync_copy(rows_v, out_hbm.at[:, wid])

    return _sc_gather


def _tc_mlp_loss(g_ref, w1_ref, b1_ref, w2_ref, b2_ref, out_ref):
    src = g_ref[0]
    tgt = g_ref[1]
    w1 = w1_ref[...]
    h = lax.dot_general(src, w1[:, :LATENT], (((1,), (1,)), ((), ())),
                        preferred_element_type=jnp.float32)
    h = h + lax.dot_general(tgt, w1[:, LATENT:], (((1,), (1,)), ((), ())),
                            preferred_element_type=jnp.float32)
    h = jnp.maximum(h + b1_ref[...], 0.0)
    s = lax.dot_general(w2_ref[...], h, (((1,), (1,)), ((), ())),
                        preferred_element_type=jnp.float32)
    s = s + b2_ref[0, 0]
    rows = lax.broadcasted_iota(jnp.int32, (1, PAD), 1)
    w = rows // CHUNK
    k = rows - w * CHUNK
    is_pos = k < POS_Q
    valid = jnp.logical_or(
        jnp.logical_and(is_pos, jnp.logical_or(w < NW - 1, k >= POS_DUP)),
        jnp.logical_and(jnp.logical_not(is_pos), w < NEG_FULL),
    )
    label = is_pos.astype(jnp.float32)
    per = jnp.maximum(s, 0.0) - s * label + jnp.log1p(jnp.exp(-jnp.abs(s)))
    per = jnp.where(valid, per, 0.0)
    out_ref[0, 0] = jnp.sum(per) * (1.0 / N_EDGE)


def kernel(v_gene, pos_edge_index, neg_edge_index, W1, b1, W2, b2):
    edges = jnp.concatenate(
        [pos_edge_index.astype(jnp.int32), neg_edge_index.astype(jnp.int32)],
        axis=1).reshape(-1)
    gathered = _make_sc_gather()(v_gene, edges)
    g = gathered.reshape(2, PAD, LATENT)

    loss = pl.pallas_call(
        _tc_mlp_loss,
        out_shape=jax.ShapeDtypeStruct((1, 1), jnp.float32),
        in_specs=[
            pl.BlockSpec(memory_space=pltpu.VMEM),
            pl.BlockSpec(memory_space=pltpu.VMEM),
            pl.BlockSpec(memory_space=pltpu.VMEM),
            pl.BlockSpec(memory_space=pltpu.VMEM),
            pl.BlockSpec(memory_space=pltpu.SMEM),
        ],
        out_specs=pl.BlockSpec(memory_space=pltpu.SMEM),
    )(g, W1, b1.reshape(1, LATENT), W2, b2.reshape(1, 1))
    return loss[0, 0]
